# Initial kernel scaffold; baseline (speedup 1.0000x reference)
import jax
import jax.numpy as jnp
from jax.experimental import pallas as pl


def kernel(x, edge_index, batch, W1, b1, g1, be1, W2, b2, g2, be2, W3, b3, M1, mb1, mg1, mbe1, M2, mb2, mg2, mbe2, M3, mb3):
    raise NotImplementedError("write your pallas kernel here")


# speedup vs baseline: 13.6158x; 13.6158x over previous
import functools

import jax
import jax.numpy as jnp
from jax import lax
from jax.experimental import pallas as pl
from jax.experimental.pallas import tpu as pltpu
from jax.experimental.pallas import tpu_sc as plsc

_N = 10000
_H = 32
_G = 128
_EPS = 1e-5
_NC = 2
_NS = 16
_NW = _NC * _NS
_CHUNK = 128
_NPAD = 10112
_RPT = _NPAD // _NS


def _make_edge_scatter(e_pad):
    ew = e_pad // _NW
    nchunk = ew // _CHUNK
    mesh = plsc.VectorSubcoreMesh(
        core_axis_name="c", subcore_axis_name="s",
        num_cores=_NC, num_subcores=_NS)

    @functools.partial(
        pl.kernel,
        mesh=mesh,
        compiler_params=pltpu.CompilerParams(use_tc_tiling_on_sc=False),
        out_type=jax.ShapeDtypeStruct((_NC, _NPAD, _H), jnp.float32),
        scratch_types=[
            pltpu.VMEM((nchunk, _CHUNK), jnp.int32),
            pltpu.VMEM((nchunk, _CHUNK), jnp.int32),
            pltpu.VMEM((_CHUNK, _H), jnp.float32),
            pltpu.VMEM_SHARED((_NPAD, _H), jnp.float32),
            pltpu.SemaphoreType.DMA,
            pltpu.SemaphoreType.DMA,
        ],
    )
    def scatter_kernel(src_hbm, dst_hbm, y_hbm, zero_hbm, out_hbm,
                       src_v, dst_v, rows_v, acc_sh, gsem, ssem):
        cid = lax.axis_index("c")
        sid = lax.axis_index("s")
        wid = cid * _NS + sid
        pltpu.sync_copy(zero_hbm.at[pl.ds(sid * _RPT, _RPT)],
                        acc_sh.at[pl.ds(sid * _RPT, _RPT)])
        pltpu.sync_copy(src_hbm.at[pl.ds(wid * nchunk, nchunk)], src_v)
        pltpu.sync_copy(dst_hbm.at[pl.ds(wid * nchunk, nchunk)], dst_v)
        plsc.subcore_barrier()

        def body(j, carry):
            pltpu.async_copy(y_hbm.at[src_v.at[j]], rows_v, gsem).wait()
            pltpu.async_copy(rows_v, acc_sh.at[dst_v.at[j]], ssem,
                             add=True).wait()
            return carry

        lax.fori_loop(0, nchunk, body, 0)
        plsc.subcore_barrier()
        pltpu.sync_copy(acc_sh.at[pl.ds(sid * _RPT, _RPT)],
                        out_hbm.at[cid, pl.ds(sid * _RPT, _RPT)])

    return scatter_kernel


def _tc_prep(x_ref, w_ref, d0_ref, d1_ref, y_ref, dinv_ref):
    deg = d0_ref[:, 0:1] + d1_ref[:, 0:1] + 1.0
    dinv = 1.0 / jnp.sqrt(deg)
    xw = jnp.dot(x_ref[...], w_ref[...], preferred_element_type=jnp.float32)
    y_ref[...] = xw * dinv
    dinv_ref[...] = dinv


def _tc_mid(a0_ref, a1_ref, y_ref, dinv_ref, b_ref, g_ref, be_ref, w_ref,
            yn_ref):
    t = (a0_ref[...] + a1_ref[...] + y_ref[...]) * dinv_ref[...] + b_ref[...]
    rows = lax.broadcasted_iota(jnp.int32, (_NPAD, 1), 0)
    mask = rows < _N
    m = jnp.sum(jnp.where(mask, t, 0.0), axis=0, keepdims=True) / _N
    d = t - m
    v = jnp.sum(jnp.where(mask, d * d, 0.0), axis=0, keepdims=True) / _N
    z = jnp.maximum(d / jnp.sqrt(v + _EPS) * g_ref[...] + be_ref[...], 0.0)
    yn_ref[...] = jnp.dot(z, w_ref[...],
                          preferred_element_type=jnp.float32) * dinv_ref[...]


def _tc_fin(a0_ref, a1_ref, y_ref, dinv_ref, b3_ref, z_ref):
    z_ref[...] = (a0_ref[...] + a1_ref[...] + y_ref[...]) * dinv_ref[...] \
        + b3_ref[...]


def kernel(x, edge_index, batch, W1, b1, g1, be1, W2, b2, g2, be2, W3, b3,
           M1, mb1, mg1, mbe1, M2, mb2, mg2, mbe2, M3, mb3):
    f32 = jnp.float32
    e = edge_index.shape[1]
    grain = _NW * _CHUNK * 8
    e_pad = ((e + grain - 1) // grain) * grain
    pad = e_pad - e
    src_p = jnp.concatenate(
        [edge_index[0], jnp.full((pad,), _N, jnp.int32)]).reshape(-1, _CHUNK)
    dst_p = jnp.concatenate(
        [edge_index[1], jnp.full((pad,), _N, jnp.int32)]).reshape(-1, _CHUNK)
    x_p = jnp.pad(x, ((0, _NPAD - _N), (0, 0)))
    zeros = jnp.zeros((_NPAD, _H), f32)
    ones = jnp.ones((_NPAD, _H), f32)

    edge_scatter = _make_edge_scatter(e_pad)

    acc_deg = edge_scatter(src_p, dst_p, ones, zeros)
    d0 = acc_deg[0, :, 0:8]
    d1 = acc_deg[1, :, 0:8]

    y1, dinv = pl.pallas_call(
        _tc_prep,
        out_shape=[jax.ShapeDtypeStruct((_NPAD, _H), f32),
                   jax.ShapeDtypeStruct((_NPAD, 1), f32)],
    )(x_p, W1, d0, d1)

    acc1 = edge_scatter(src_p, dst_p, y1, zeros)
    y2 = pl.pallas_call(
        _tc_mid,
        out_shape=jax.ShapeDtypeStruct((_NPAD, _H), f32),
    )(acc1[0], acc1[1], y1, dinv, b1.reshape(1, -1), g1.reshape(1, -1),
      be1.reshape(1, -1), W2)

    acc2 = edge_scatter(src_p, dst_p, y2, zeros)
    y3 = pl.pallas_call(
        _tc_mid,
        out_shape=jax.ShapeDtypeStruct((_NPAD, _H), f32),
    )(acc2[0], acc2[1], y2, dinv, b2.reshape(1, -1), g2.reshape(1, -1),
      be2.reshape(1, -1), W3)

    acc3 = edge_scatter(src_p, dst_p, y3, zeros)
    z3 = pl.pallas_call(
        _tc_fin,
        out_shape=jax.ShapeDtypeStruct((_NPAD, _H), f32),
    )(acc3[0], acc3[1], y3, dinv, b3.reshape(1, -1))[: _N]

    sums = jax.ops.segment_sum(z3, batch, num_segments=_G)
    cnts = jax.ops.segment_sum(jnp.ones((_N, 1), f32), batch, num_segments=_G)
    h = sums / jnp.maximum(cnts, 1.0)

    def bn(t, g, be):
        m = t.mean(axis=0)
        v = t.var(axis=0)
        return (t - m) / jnp.sqrt(v + _EPS) * g + be

    h = jax.nn.relu(bn(h @ M1 + mb1, mg1, mbe1))
    h = jax.nn.relu(bn(h @ M2 + mb2, mg2, mbe2))
    return h @ M3 + mb3
